# Initial kernel scaffold; baseline (speedup 1.0000x reference)
import jax
import jax.numpy as jnp
from jax.experimental import pallas as pl


def kernel(x, state, A0, A1, W_gate, b_gate, W_update, b_update):
    raise NotImplementedError("write your pallas kernel here")


# speedup vs baseline: 1.4962x; 1.4962x over previous
import jax
import jax.numpy as jnp
from jax.experimental import pallas as pl

NODES = 4096
HID = 32
NB = 4
CIN = HID + 1
WID = NB * CIN
SWID = NB * HID
ROWS = 512


def _mm_kernel(a_ref, x_ref, o_ref):
    o_ref[...] = jnp.dot(a_ref[...], x_ref[...],
                         preferred_element_type=jnp.float32
                         ).astype(jnp.bfloat16)


def _mm(A, X):
    n, w = NODES, X.shape[1]
    return pl.pallas_call(
        _mm_kernel,
        grid=(n // ROWS,),
        in_specs=[
            pl.BlockSpec((ROWS, n), lambda i: (i, 0)),
            pl.BlockSpec((n, w), lambda i: (0, 0)),
        ],
        out_specs=pl.BlockSpec((ROWS, w), lambda i: (i, 0)),
        out_shape=jax.ShapeDtypeStruct((n, w), jnp.bfloat16),
    )(A, X)


def _mm_cast_kernel(a_ref, x_ref, o_ref, ab_ref):
    ab = a_ref[...].astype(jnp.bfloat16)
    ab_ref[...] = ab
    o_ref[...] = jnp.dot(ab, x_ref[...],
                         preferred_element_type=jnp.float32
                         ).astype(jnp.bfloat16)


def _mm_cast(A, X):
    n, w = NODES, X.shape[1]
    return pl.pallas_call(
        _mm_cast_kernel,
        grid=(n // ROWS,),
        in_specs=[
            pl.BlockSpec((ROWS, n), lambda i: (i, 0)),
            pl.BlockSpec((n, w), lambda i: (0, 0)),
        ],
        out_specs=[pl.BlockSpec((ROWS, w), lambda i: (i, 0)),
                   pl.BlockSpec((ROWS, n), lambda i: (i, 0))],
        out_shape=(jax.ShapeDtypeStruct((n, w), jnp.bfloat16),
                   jax.ShapeDtypeStruct((n, n), jnp.bfloat16)),
    )(A, X)


def _lin5(terms, last, w_ref, b_ref):
    acc = b_ref[...].astype(jnp.float32)
    for p, t in enumerate(terms):
        acc = acc + jnp.dot(t[...].astype(jnp.float32),
                            w_ref[p * WID:(p + 1) * WID, :],
                            preferred_element_type=jnp.float32)
    return acc + jnp.dot(last, w_ref[4 * WID:5 * WID, :],
                         preferred_element_type=jnp.float32)


def _gate_kernel(a_ref, tfull_ref, y_ref, t0_ref, u0_ref, t1_ref, x_ref,
                 s_ref, w_ref, b_ref, c_ref, r_ref):
    u1 = jnp.dot(a_ref[...], tfull_ref[...],
                 preferred_element_type=jnp.float32)
    zr = jax.nn.sigmoid(
        _lin5((y_ref, t0_ref, u0_ref, t1_ref), u1, w_ref, b_ref))
    z = zr[:, :SWID]
    r_ref[...] = zr[:, SWID:]
    c_ref[...] = jnp.concatenate(
        [x_ref[...], z * s_ref[...]], axis=1).astype(jnp.bfloat16)


def _final_kernel(a_ref, tfull_ref, c_ref, t0_ref, u0_ref, t1_ref, s_ref,
                  r_ref, w_ref, b_ref, h_ref):
    u1 = jnp.dot(a_ref[...], tfull_ref[...],
                 preferred_element_type=jnp.float32)
    hc = jnp.tanh(
        _lin5((c_ref, t0_ref, u0_ref, t1_ref), u1, w_ref, b_ref))
    r = r_ref[...]
    h_ref[...] = r * s_ref[...] + (1.0 - r) * hc


def _expand_w(W5):
    O = W5.shape[-1]
    eye = jnp.eye(NB, dtype=W5.dtype)
    xpart = jnp.einsum('ib,po->pibo', eye, W5[:, 0, :])
    spart = jnp.einsum('bc,pho->pbhco', eye, W5[:, 1:, :])
    xpart = xpart.reshape(5, NB, NB * O)
    spart = spart.reshape(5, NB * HID, NB * O)
    return jnp.concatenate([xpart, spart], axis=1).reshape(5 * WID, NB * O)


def _row_spec(width):
    return pl.BlockSpec((ROWS, width), lambda i: (i, 0))


def _full_spec(shape):
    return pl.BlockSpec(shape, lambda i: (0, 0))


def kernel(x, state, A0, A1, W_gate, b_gate, W_update, b_update):
    f32 = jnp.float32
    xT = x[:, :, 0].T
    sT = state.transpose(1, 0, 2).reshape(NODES, SWID)
    Y = jnp.concatenate([xT, sT], axis=1)

    W5g = W_gate.reshape(5, CIN, 2 * HID)
    Wz = _expand_w(W5g[:, :, :HID])
    Wr = _expand_w(W5g[:, :, HID:])
    Wg = jnp.concatenate([Wz, Wr], axis=1)
    bg = jnp.concatenate([jnp.tile(b_gate[:HID], NB),
                          jnp.tile(b_gate[HID:], NB)]).reshape(1, 2 * SWID)
    Wu = _expand_w(W_update.reshape(5, CIN, HID))
    bu = jnp.tile(b_update, NB).reshape(1, SWID)

    bf16 = jnp.bfloat16
    Yb = Y.astype(bf16)

    T0, A0b = _mm_cast(A0, Yb)
    U0 = _mm(A0b, T0)
    T1, A1b = _mm_cast(A1, Yb)

    C, R = pl.pallas_call(
        _gate_kernel,
        grid=(NODES // ROWS,),
        in_specs=[pl.BlockSpec((ROWS, NODES), lambda i: (i, 0)),
                  _full_spec((NODES, WID))] +
                 [_row_spec(WID)] * 4 +
                 [_row_spec(NB), _row_spec(SWID),
                  _full_spec((5 * WID, 2 * SWID)), _full_spec((1, 2 * SWID))],
        out_specs=[_row_spec(WID), _row_spec(SWID)],
        out_shape=(jax.ShapeDtypeStruct((NODES, WID), bf16),
                   jax.ShapeDtypeStruct((NODES, SWID), f32)),
    )(A1b, T1, Yb, T0, U0, T1, xT, sT, Wg, bg)

    T0c = _mm(A0b, C)
    U0c = _mm(A0b, T0c)
    T1c = _mm(A1b, C)

    H = pl.pallas_call(
        _final_kernel,
        grid=(NODES // ROWS,),
        in_specs=[pl.BlockSpec((ROWS, NODES), lambda i: (i, 0)),
                  _full_spec((NODES, WID))] +
                 [_row_spec(WID)] * 4 +
                 [_row_spec(SWID), _row_spec(SWID),
                  _full_spec((5 * WID, SWID)), _full_spec((1, SWID))],
        out_specs=_row_spec(SWID),
        out_shape=jax.ShapeDtypeStruct((NODES, SWID), f32),
    )(A1b, T1c, C, T0c, U0c, T1c, sT, R, Wu, bu)

    return H.reshape(NODES, NB, HID).transpose(1, 0, 2)


# speedup vs baseline: 1.5346x; 1.0257x over previous
import jax
import jax.numpy as jnp
from jax.experimental import pallas as pl

NODES = 4096
HID = 32
NB = 4
CIN = HID + 1
WID = NB * CIN
SWID = NB * HID
ROWS = 1024
CROWS = 512


def _mm_kernel(a_ref, x_ref, o_ref):
    o_ref[...] = jnp.dot(a_ref[...], x_ref[...],
                         preferred_element_type=jnp.float32
                         ).astype(jnp.bfloat16)


def _mm(A, X):
    n, w = NODES, X.shape[1]
    return pl.pallas_call(
        _mm_kernel,
        grid=(n // ROWS,),
        in_specs=[
            pl.BlockSpec((ROWS, n), lambda i: (i, 0)),
            pl.BlockSpec((n, w), lambda i: (0, 0)),
        ],
        out_specs=pl.BlockSpec((ROWS, w), lambda i: (i, 0)),
        out_shape=jax.ShapeDtypeStruct((n, w), jnp.bfloat16),
    )(A, X)


def _mm_cast_kernel(a_ref, x_ref, o_ref, ab_ref):
    ab = a_ref[...].astype(jnp.bfloat16)
    ab_ref[...] = ab
    o_ref[...] = jnp.dot(ab, x_ref[...],
                         preferred_element_type=jnp.float32
                         ).astype(jnp.bfloat16)


def _mm_cast(A, X):
    n, w = NODES, X.shape[1]
    return pl.pallas_call(
        _mm_cast_kernel,
        grid=(n // CROWS,),
        in_specs=[
            pl.BlockSpec((CROWS, n), lambda i: (i, 0)),
            pl.BlockSpec((n, w), lambda i: (0, 0)),
        ],
        out_specs=[pl.BlockSpec((CROWS, w), lambda i: (i, 0)),
                   pl.BlockSpec((CROWS, n), lambda i: (i, 0))],
        out_shape=(jax.ShapeDtypeStruct((n, w), jnp.bfloat16),
                   jax.ShapeDtypeStruct((n, n), jnp.bfloat16)),
    )(A, X)


def _lin5(terms, last, w_ref, b_ref):
    acc = b_ref[...].astype(jnp.float32)
    for p, t in enumerate(terms):
        acc = acc + jnp.dot(t[...].astype(jnp.float32),
                            w_ref[p * WID:(p + 1) * WID, :],
                            preferred_element_type=jnp.float32)
    return acc + jnp.dot(last, w_ref[4 * WID:5 * WID, :],
                         preferred_element_type=jnp.float32)


def _gate_kernel(a_ref, tfull_ref, y_ref, t0_ref, u0_ref, t1_ref, x_ref,
                 s_ref, w_ref, b_ref, c_ref, r_ref):
    u1 = jnp.dot(a_ref[...], tfull_ref[...],
                 preferred_element_type=jnp.float32)
    zr = jax.nn.sigmoid(
        _lin5((y_ref, t0_ref, u0_ref, t1_ref), u1, w_ref, b_ref))
    z = zr[:, :SWID]
    r_ref[...] = zr[:, SWID:]
    c_ref[...] = jnp.concatenate(
        [x_ref[...], z * s_ref[...]], axis=1).astype(jnp.bfloat16)


def _final_kernel(a_ref, tfull_ref, c_ref, t0_ref, u0_ref, t1_ref, s_ref,
                  r_ref, w_ref, b_ref, h_ref):
    u1 = jnp.dot(a_ref[...], tfull_ref[...],
                 preferred_element_type=jnp.float32)
    hc = jnp.tanh(
        _lin5((c_ref, t0_ref, u0_ref, t1_ref), u1, w_ref, b_ref))
    r = r_ref[...]
    h_ref[...] = r * s_ref[...] + (1.0 - r) * hc


def _expand_w(W5):
    O = W5.shape[-1]
    eye = jnp.eye(NB, dtype=W5.dtype)
    xpart = jnp.einsum('ib,po->pibo', eye, W5[:, 0, :])
    spart = jnp.einsum('bc,pho->pbhco', eye, W5[:, 1:, :])
    xpart = xpart.reshape(5, NB, NB * O)
    spart = spart.reshape(5, NB * HID, NB * O)
    return jnp.concatenate([xpart, spart], axis=1).reshape(5 * WID, NB * O)


def _row_spec(width):
    return pl.BlockSpec((ROWS, width), lambda i: (i, 0))


def _full_spec(shape):
    return pl.BlockSpec(shape, lambda i: (0, 0))


def kernel(x, state, A0, A1, W_gate, b_gate, W_update, b_update):
    f32 = jnp.float32
    xT = x[:, :, 0].T
    sT = state.transpose(1, 0, 2).reshape(NODES, SWID)
    Y = jnp.concatenate([xT, sT], axis=1)

    W5g = W_gate.reshape(5, CIN, 2 * HID)
    Wz = _expand_w(W5g[:, :, :HID])
    Wr = _expand_w(W5g[:, :, HID:])
    Wg = jnp.concatenate([Wz, Wr], axis=1)
    bg = jnp.concatenate([jnp.tile(b_gate[:HID], NB),
                          jnp.tile(b_gate[HID:], NB)]).reshape(1, 2 * SWID)
    Wu = _expand_w(W_update.reshape(5, CIN, HID))
    bu = jnp.tile(b_update, NB).reshape(1, SWID)

    bf16 = jnp.bfloat16
    Yb = Y.astype(bf16)

    T0, A0b = _mm_cast(A0, Yb)
    U0 = _mm(A0b, T0)
    T1, A1b = _mm_cast(A1, Yb)

    C, R = pl.pallas_call(
        _gate_kernel,
        grid=(NODES // ROWS,),
        in_specs=[pl.BlockSpec((ROWS, NODES), lambda i: (i, 0)),
                  _full_spec((NODES, WID))] +
                 [_row_spec(WID)] * 4 +
                 [_row_spec(NB), _row_spec(SWID),
                  _full_spec((5 * WID, 2 * SWID)), _full_spec((1, 2 * SWID))],
        out_specs=[_row_spec(WID), _row_spec(SWID)],
        out_shape=(jax.ShapeDtypeStruct((NODES, WID), bf16),
                   jax.ShapeDtypeStruct((NODES, SWID), f32)),
    )(A1b, T1, Yb, T0, U0, T1, xT, sT, Wg, bg)

    T0c = _mm(A0b, C)
    U0c = _mm(A0b, T0c)
    T1c = _mm(A1b, C)

    H = pl.pallas_call(
        _final_kernel,
        grid=(NODES // ROWS,),
        in_specs=[pl.BlockSpec((ROWS, NODES), lambda i: (i, 0)),
                  _full_spec((NODES, WID))] +
                 [_row_spec(WID)] * 4 +
                 [_row_spec(SWID), _row_spec(SWID),
                  _full_spec((5 * WID, SWID)), _full_spec((1, SWID))],
        out_specs=_row_spec(SWID),
        out_shape=jax.ShapeDtypeStruct((NODES, SWID), f32),
    )(A1b, T1c, C, T0c, U0c, T1c, sT, R, Wu, bu)

    return H.reshape(NODES, NB, HID).transpose(1, 0, 2)
